# baseline (speedup 1.0000x reference)
import jax
import jax.numpy as jnp
from jax import lax
from jax.experimental import pallas as pl
from jax.experimental.pallas import tpu as pltpu
from jax.experimental.pallas import tpu_sc as plsc

_NV = 10000
_NC = 42000
_K = 3
_B = 128

_NSUB = 16
_CPT = 2624
_CPT_LAST = _NC - (_NSUB - 1) * _CPT
_WPT = _CPT * _K
_BUFW = _CPT_LAST * _K
_SIGN = -2**31
_IMASK = 0x7FFFFFFF


def _sat_body(x_hbm, pk_hbm, out_hbm,
              x_v, buf_v, acc_v, shared, red_v, out_v, sem1, sem2):
    s = lax.axis_index("s")

    cp1 = pltpu.async_copy(x_hbm, x_v, sem1)
    cp2 = pltpu.async_copy(pk_hbm.at[pl.ds(s * _WPT, _BUFW)], buf_v, sem2)
    cp1.wait()
    cp2.wait()

    i3 = lax.iota(jnp.int32, 16) * 3

    def body(t, acc):
        base = t * (16 * _K)
        pre = jnp.full((16,), float(_K - 1), dtype=jnp.float32)
        for j in range(_K):
            pk = plsc.load_gather(buf_v, [i3 + (base + j)])
            ij = pk & jnp.int32(_IMASK)
            v = plsc.load_gather(x_v, [ij])
            xv = plsc.bitcast(
                plsc.bitcast(v, jnp.int32) ^ (pk & jnp.int32(_SIGN)),
                jnp.float32)
            pre = pre + jnp.sign(xv)
        return acc + jnp.sign(pre)

    trip = jnp.where(s == _NSUB - 1, _CPT_LAST // 16, _CPT // 16)
    acc = lax.fori_loop(0, trip, body, jnp.zeros((16,), jnp.float32))

    acc_v[...] = acc
    pltpu.sync_copy(acc_v, shared.at[s])
    plsc.subcore_barrier()

    @pl.when(s == 0)
    def _():
        pltpu.sync_copy(shared, red_v)
        tot = jnp.zeros((16,), jnp.float32)
        for si in range(_NSUB):
            tot = tot + red_v[si]
        total = jnp.sum(tot)
        outvec = jnp.sign(jnp.broadcast_to(total - float(_NC - 1), (16,)))
        for k in range(_B // 16):
            out_v[pl.ds(k * 16, 16)] = outvec
        pltpu.sync_copy(out_v, out_hbm)


def kernel(input, emb_weight, or_weight, clause_idx):
    del input
    x_flat = emb_weight.reshape(-1)
    packed = (clause_idx.reshape(-1)
              | (lax.bitcast_convert_type(or_weight.reshape(-1), jnp.int32)
                 & jnp.int32(_SIGN)))

    mesh = plsc.VectorSubcoreMesh(
        core_axis_name="c", subcore_axis_name="s", num_cores=1)
    f = pl.kernel(
        _sat_body,
        mesh=mesh,
        out_type=jax.ShapeDtypeStruct((_B,), jnp.float32),
        compiler_params=pltpu.CompilerParams(needs_layout_passes=False),
        scratch_types=[
            pltpu.VMEM((_NV,), jnp.float32),
            pltpu.VMEM((_BUFW,), jnp.int32),
            pltpu.VMEM((16,), jnp.float32),
            pltpu.VMEM_SHARED((_NSUB, 16), jnp.float32),
            pltpu.VMEM((_NSUB, 16), jnp.float32),
            pltpu.VMEM((_B,), jnp.float32),
            pltpu.SemaphoreType.DMA,
            pltpu.SemaphoreType.DMA,
        ],
    )
    return f(x_flat, packed)


# speedup vs baseline: 1.9606x; 1.9606x over previous
import jax
import jax.numpy as jnp
from jax import lax
from jax.experimental import pallas as pl
from jax.experimental.pallas import tpu as pltpu
from jax.experimental.pallas import tpu_sc as plsc

_NV = 10000
_NC = 42000
_K = 3
_B = 128

_NSUB = 16
_CPT = 2640
_NCP = _NSUB * _CPT
_STEPS = _CPT // 16
_PAD = _NCP - _NC
_BUFW = _K * _CPT
_H0 = 1312
_H1 = _CPT - _H0
_H0W = _K * _H0
_H1W = _K * _H1

_SIGN = -2**31
_IMASK = 0x7FFFFFFF
_BIG = 1.7e38


def _sat_body(x_hbm, pk_hbm, out_hbm,
              x_v, buf_v, acc_v, shared, red_v, out_v, sem1, sem2, sem3):
    s = lax.axis_index("s")

    base = s * _BUFW
    cp1 = pltpu.async_copy(x_hbm, x_v, sem1)
    cp2 = pltpu.async_copy(pk_hbm.at[pl.ds(base, _H0W)],
                           buf_v.at[pl.ds(0, _H0W)], sem2)
    cp3 = pltpu.async_copy(pk_hbm.at[pl.ds(base + _H0W, _H1W)],
                           buf_v.at[pl.ds(_H0W, _H1W)], sem3)
    cp1.wait()
    cp2.wait()

    def make_body(org, cpt):
        def body(t, acc):
            off = t * 16
            pre = jnp.full((16,), float(_K - 1), dtype=jnp.float32)
            for j in range(_K):
                pk = buf_v[pl.ds(org + j * cpt + off, 16)]
                v = plsc.load_gather(x_v, [pk & jnp.int32(_IMASK)])
                xv = plsc.bitcast(
                    plsc.bitcast(v, jnp.int32) ^ (pk & jnp.int32(_SIGN)),
                    jnp.float32)
                t_j = jnp.minimum(jnp.maximum(xv * _BIG * _BIG, -1.0), 1.0)
                pre = pre + t_j
            return acc + jnp.minimum(pre, 1.0)
        return body

    acc = lax.fori_loop(0, _H0 // 16, make_body(0, _H0),
                        jnp.zeros((16,), jnp.float32))
    cp3.wait()
    acc = lax.fori_loop(0, _H1 // 16, make_body(_H0W, _H1), acc)

    acc_v[...] = acc
    pltpu.sync_copy(acc_v, shared.at[s])
    plsc.subcore_barrier()

    @pl.when(s == 0)
    def _():
        pltpu.sync_copy(shared, red_v)
        tot = jnp.zeros((16,), jnp.float32)
        for si in range(_NSUB):
            tot = tot + red_v[si]
        total = jnp.sum(tot)
        thresh = float(_PAD + _NC - 1)
        outvec = jnp.sign(jnp.broadcast_to(total - thresh, (16,)))
        for k in range(_B // 16):
            out_v[pl.ds(k * 16, 16)] = outvec
        pltpu.sync_copy(out_v, out_hbm)


def kernel(input, emb_weight, or_weight, clause_idx):
    del input
    x_flat = emb_weight.reshape(-1)
    packed = (clause_idx
              | (lax.bitcast_convert_type(or_weight, jnp.int32)
                 & jnp.int32(_SIGN)))
    pad_row = jnp.array([[0, _SIGN, 0]], dtype=jnp.int32)
    packed = jnp.concatenate(
        [packed, jnp.broadcast_to(pad_row, (_PAD, _K))], axis=0)
    p = packed.reshape(_NSUB, _CPT, _K)
    a = p[:, :_H0, :].transpose(0, 2, 1).reshape(_NSUB, _H0W)
    b = p[:, _H0:, :].transpose(0, 2, 1).reshape(_NSUB, _H1W)
    buf = jnp.concatenate([a, b], axis=1).reshape(-1)

    mesh = plsc.VectorSubcoreMesh(
        core_axis_name="c", subcore_axis_name="s", num_cores=1)
    f = pl.kernel(
        _sat_body,
        mesh=mesh,
        out_type=jax.ShapeDtypeStruct((_B,), jnp.float32),
        compiler_params=pltpu.CompilerParams(needs_layout_passes=False),
        scratch_types=[
            pltpu.VMEM((_NV,), jnp.float32),
            pltpu.VMEM((_BUFW,), jnp.int32),
            pltpu.VMEM((16,), jnp.float32),
            pltpu.VMEM_SHARED((_NSUB, 16), jnp.float32),
            pltpu.VMEM((_NSUB, 16), jnp.float32),
            pltpu.VMEM((_B,), jnp.float32),
            pltpu.SemaphoreType.DMA,
            pltpu.SemaphoreType.DMA,
            pltpu.SemaphoreType.DMA,
        ],
    )
    return f(x_flat, buf)


# speedup vs baseline: 2.1188x; 1.0807x over previous
import jax
import jax.numpy as jnp
from jax import lax
from jax.experimental import pallas as pl
from jax.experimental.pallas import tpu as pltpu
from jax.experimental.pallas import tpu_sc as plsc

_NV = 10000
_NC = 42000
_K = 3
_B = 128

_NSUB = 16
_CPT = 2640
_NCP = _NSUB * _CPT
_STEPS = _CPT // 16
_PAD = _NCP - _NC
_BUFW = _K * _CPT

_SIGN = -2**31
_IMASK = 0x7FFFFFFF
_BIG = 1.7e38


def _sat_body(x_hbm, pk_hbm, out_hbm,
              x_v, buf_v, acc_v, shared, red_v, out_v, sem1, sem2):
    s = lax.axis_index("s")

    cp1 = pltpu.async_copy(x_hbm, x_v, sem1)
    cp2 = pltpu.async_copy(pk_hbm.at[pl.ds(s * _BUFW, _BUFW)], buf_v, sem2)
    cp1.wait()
    cp2.wait()

    def body(t, acc):
        off = t * 16
        pre = jnp.full((16,), float(_K - 1), dtype=jnp.float32)
        for j in range(_K):
            pk = buf_v[pl.ds(j * _CPT + off, 16)]
            v = plsc.load_gather(x_v, [pk & jnp.int32(_IMASK)])
            xv = plsc.bitcast(
                plsc.bitcast(v, jnp.int32) ^ (pk & jnp.int32(_SIGN)),
                jnp.float32)
            t_j = jnp.minimum(jnp.maximum(xv * _BIG * _BIG, -1.0), 1.0)
            pre = pre + t_j
        return acc + jnp.minimum(pre, 1.0)

    acc = lax.fori_loop(0, _STEPS, body, jnp.zeros((16,), jnp.float32))

    acc_v[...] = acc
    pltpu.sync_copy(acc_v, shared.at[s])
    plsc.subcore_barrier()

    @pl.when(s == 0)
    def _():
        pltpu.sync_copy(shared, red_v)
        tot = jnp.zeros((16,), jnp.float32)
        for si in range(_NSUB):
            tot = tot + red_v[si]
        total = jnp.sum(tot)
        thresh = float(_PAD + _NC - 1)
        outvec = jnp.sign(jnp.broadcast_to(total - thresh, (16,)))
        for k in range(_B // 16):
            out_v[pl.ds(k * 16, 16)] = outvec
        pltpu.sync_copy(out_v, out_hbm)


def kernel(input, emb_weight, or_weight, clause_idx):
    del input
    x_flat = emb_weight.reshape(-1)
    packed = (clause_idx
              | (lax.bitcast_convert_type(or_weight, jnp.int32)
                 & jnp.int32(_SIGN)))
    pad_row = jnp.array([[0, _SIGN, 0]], dtype=jnp.int32)
    packed = jnp.concatenate(
        [packed, jnp.broadcast_to(pad_row, (_PAD, _K))], axis=0)
    buf = packed.reshape(_NSUB, _CPT, _K).transpose(0, 2, 1).reshape(-1)

    mesh = plsc.VectorSubcoreMesh(
        core_axis_name="c", subcore_axis_name="s", num_cores=1)
    f = pl.kernel(
        _sat_body,
        mesh=mesh,
        out_type=jax.ShapeDtypeStruct((_B,), jnp.float32),
        compiler_params=pltpu.CompilerParams(needs_layout_passes=False),
        scratch_types=[
            pltpu.VMEM((_NV,), jnp.float32),
            pltpu.VMEM((_BUFW,), jnp.int32),
            pltpu.VMEM((16,), jnp.float32),
            pltpu.VMEM_SHARED((_NSUB, 16), jnp.float32),
            pltpu.VMEM((_NSUB, 16), jnp.float32),
            pltpu.VMEM((_B,), jnp.float32),
            pltpu.SemaphoreType.DMA,
            pltpu.SemaphoreType.DMA,
        ],
    )
    return f(x_flat, buf)
